# baseline (speedup 1.0000x reference)
import jax
import jax.numpy as jnp
from jax import lax
from jax.experimental import pallas as pl
from jax.experimental.pallas import tpu as pltpu
from jax.experimental.pallas import tpu_sc as plsc

R, C = 16384, 1024
SC_ROWS = 7168
TC_ROWS = R - SC_ROWS
NW = 32
ROWS_W = SC_ROWS // NW
CR = 8
NCH = ROWS_W // CR
NBUF = 4
NOUTER = NCH // NBUF
TBL = 128
L = 16
VPC = CR * C // L
TC_BR = 1024


def _sc_body(dist_hbm, grp_hbm, tbl_hbm, out_hbm,
          tbl_v, dist_v0, dist_v1, dist_v2, dist_v3,
          grp_v0, grp_v1, grp_v2, grp_v3,
          out_v0, out_v1, out_v2, out_v3,
          sd0, sd1, sd2, sd3, sg0, sg1, sg2, sg3, so0, so1, so2, so3):
  dist_bufs = [dist_v0, dist_v1, dist_v2, dist_v3]
  grp_bufs = [grp_v0, grp_v1, grp_v2, grp_v3]
  out_bufs = [out_v0, out_v1, out_v2, out_v3]
  cid = lax.axis_index("c")
  sid = lax.axis_index("s")
  wid = sid * 2 + cid
  base = wid * ROWS_W

  in_sems_d = [sd0, sd1, sd2, sd3]
  in_sems_g = [sg0, sg1, sg2, sg3]
  out_sems = [so0, so1, so2, so3]

  pltpu.sync_copy(tbl_hbm, tbl_v)

  lanes = lax.iota(jnp.int32, L)

  def in_slices(cg):
    row = base + cg * CR
    return dist_hbm.at[pl.ds(row, CR)], grp_hbm.at[pl.ds(row, CR)]

  def out_slice(cg):
    return out_hbm.at[pl.ds(base + cg * CR, CR)]

  for b in range(NBUF):
    dsl, gsl = in_slices(b)
    pltpu.async_copy(dsl, dist_bufs[b], in_sems_d[b])
    pltpu.async_copy(gsl, grp_bufs[b], in_sems_g[b])

  @pl.loop(0, NOUTER)
  def outer(j):
    for b in range(NBUF):
      cg = j * NBUF + b
      dsl, gsl = in_slices(cg)
      pltpu.make_async_copy(dsl, dist_bufs[b], in_sems_d[b]).wait()
      pltpu.make_async_copy(gsl, grp_bufs[b], in_sems_g[b]).wait()

      @pl.when(j > 0)
      def _():
        pltpu.make_async_copy(out_bufs[b], out_slice((j - 1) * NBUF + b),
                              out_sems[b]).wait()

      db = dist_bufs[b]
      gb = grp_bufs[b]
      ob = out_bufs[b]

      @plsc.parallel_loop(0, VPC, 1, unroll=8)
      def inner(i):
        r = i >> 6
        c = (i & 63) << 4
        g = gb[r, pl.ds(c, L)]
        w = plsc.load_gather(tbl_v, [g])
        d = db[r, pl.ds(c, L)]
        ob[r, pl.ds(c, L)] = jnp.maximum(d * w, 0.0)

      pltpu.async_copy(ob, out_slice(cg), out_sems[b])

      @pl.when(j < NOUTER - 1)
      def _():
        dsl2, gsl2 = in_slices(cg + NBUF)
        pltpu.async_copy(dsl2, dist_bufs[b], in_sems_d[b])
        pltpu.async_copy(gsl2, grp_bufs[b], in_sems_g[b])

  for b in range(NBUF):
    pltpu.make_async_copy(out_bufs[b], out_slice((NOUTER - 1) * NBUF + b),
                          out_sems[b]).wait()


def _sc_run(dist, grp, tbl):
  mesh = plsc.VectorSubcoreMesh(core_axis_name="c", subcore_axis_name="s")
  return pl.kernel(
      _sc_body,
      out_type=jax.ShapeDtypeStruct((SC_ROWS, C), jnp.float32),
      mesh=mesh,
      compiler_params=pltpu.CompilerParams(needs_layout_passes=False),
      scratch_types=[
          pltpu.VMEM((TBL,), jnp.float32),
          pltpu.VMEM((CR, C), jnp.float32),
          pltpu.VMEM((CR, C), jnp.float32),
          pltpu.VMEM((CR, C), jnp.float32),
          pltpu.VMEM((CR, C), jnp.float32),
          pltpu.VMEM((CR, C), jnp.int32),
          pltpu.VMEM((CR, C), jnp.int32),
          pltpu.VMEM((CR, C), jnp.int32),
          pltpu.VMEM((CR, C), jnp.int32),
          pltpu.VMEM((CR, C), jnp.float32),
          pltpu.VMEM((CR, C), jnp.float32),
          pltpu.VMEM((CR, C), jnp.float32),
          pltpu.VMEM((CR, C), jnp.float32),          pltpu.SemaphoreType.DMA,
          pltpu.SemaphoreType.DMA,
          pltpu.SemaphoreType.DMA,
          pltpu.SemaphoreType.DMA,
          pltpu.SemaphoreType.DMA,
          pltpu.SemaphoreType.DMA,
          pltpu.SemaphoreType.DMA,
          pltpu.SemaphoreType.DMA,
          pltpu.SemaphoreType.DMA,
          pltpu.SemaphoreType.DMA,
          pltpu.SemaphoreType.DMA,
          pltpu.SemaphoreType.DMA,
      ],
  )(dist, grp, tbl)


def _tc_body(tbl_ref, dist_ref, grp_ref, out_ref):
  tb = jnp.broadcast_to(tbl_ref[0:1, :], (TC_BR, TBL))
  g = grp_ref[...]
  w = jnp.take_along_axis(tb, g, axis=1, mode="promise_in_bounds")
  out_ref[...] = jnp.maximum(dist_ref[...] * w, 0.0)


def _tc_run(dist, grp, tbl2d):
  off = SC_ROWS // TC_BR
  return pl.pallas_call(
      _tc_body,
      grid=(TC_ROWS // TC_BR,),
      in_specs=[
          pl.BlockSpec((8, TBL), lambda i: (0, 0)),
          pl.BlockSpec((TC_BR, C), lambda i: (off + i, 0)),
          pl.BlockSpec((TC_BR, C), lambda i: (off + i, 0)),
      ],
      out_specs=pl.BlockSpec((TC_BR, C), lambda i: (off + i, 0)),
      out_shape=jax.ShapeDtypeStruct((R, C), jnp.float32),
  )(tbl2d, dist, grp)


@jax.jit
def _run(dist, grp, tbl):
  sc_part = _sc_run(dist, grp, tbl)
  tc_full = _tc_run(dist, grp, jnp.broadcast_to(tbl[None, :], (8, TBL)))
  return lax.dynamic_update_slice(tc_full, sc_part, (0, 0))


def kernel(distance_matrix, groups, group_weights):
  tbl = jnp.zeros((TBL,), jnp.float32).at[:group_weights.shape[0]].set(
      group_weights[:, 0])
  return _run(distance_matrix, groups.astype(jnp.int32), tbl)


# speedup vs baseline: 1.1681x; 1.1681x over previous
import jax
import jax.numpy as jnp
from jax import lax
from jax.experimental import pallas as pl
from jax.experimental.pallas import tpu as pltpu
from jax.experimental.pallas import tpu_sc as plsc

R, C = 16384, 1024
NW = 32
ROWS_W = R // NW
CR = 4
NCH = ROWS_W // CR
NBUF = 8
NOUTER = NCH // NBUF
TBL = 128
L = 16
VPC = CR * C // L


def _body(dist_hbm, grp_hbm, tbl_hbm, out_hbm, tbl_v, *scr):
  dist_bufs = list(scr[0:NBUF])
  grp_bufs = list(scr[NBUF:2 * NBUF])
  out_bufs = list(scr[2 * NBUF:3 * NBUF])
  cid = lax.axis_index("c")
  sid = lax.axis_index("s")
  wid = sid * 2 + cid
  base = wid * ROWS_W

  in_sems_d = list(scr[3 * NBUF:4 * NBUF])
  in_sems_g = list(scr[4 * NBUF:5 * NBUF])
  out_sems = list(scr[5 * NBUF:6 * NBUF])

  pltpu.sync_copy(tbl_hbm, tbl_v)

  lanes = lax.iota(jnp.int32, L)

  def in_slices(cg):
    row = base + cg * CR
    return dist_hbm.at[pl.ds(row, CR)], grp_hbm.at[pl.ds(row, CR)]

  def out_slice(cg):
    return out_hbm.at[pl.ds(base + cg * CR, CR)]

  for b in range(NBUF):
    dsl, gsl = in_slices(b)
    pltpu.async_copy(dsl, dist_bufs[b], in_sems_d[b])
    pltpu.async_copy(gsl, grp_bufs[b], in_sems_g[b])

  @pl.loop(0, NOUTER)
  def outer(j):
    for b in range(NBUF):
      cg = j * NBUF + b
      dsl, gsl = in_slices(cg)
      pltpu.make_async_copy(dsl, dist_bufs[b], in_sems_d[b]).wait()
      pltpu.make_async_copy(gsl, grp_bufs[b], in_sems_g[b]).wait()

      @pl.when(j > 0)
      def _():
        pltpu.make_async_copy(out_bufs[b], out_slice((j - 1) * NBUF + b),
                              out_sems[b]).wait()

      db = dist_bufs[b]
      gb = grp_bufs[b]
      ob = out_bufs[b]

      @plsc.parallel_loop(0, VPC, 1, unroll=8)
      def inner(i):
        r = i >> 6
        c = (i & 63) << 4
        g = gb[r, pl.ds(c, L)]
        w = plsc.load_gather(tbl_v, [g])
        d = db[r, pl.ds(c, L)]
        ob[r, pl.ds(c, L)] = jnp.maximum(d * w, 0.0)

      pltpu.async_copy(ob, out_slice(cg), out_sems[b])

      @pl.when(j < NOUTER - 1)
      def _():
        dsl2, gsl2 = in_slices(cg + NBUF)
        pltpu.async_copy(dsl2, dist_bufs[b], in_sems_d[b])
        pltpu.async_copy(gsl2, grp_bufs[b], in_sems_g[b])

  for b in range(NBUF):
    pltpu.make_async_copy(out_bufs[b], out_slice((NOUTER - 1) * NBUF + b),
                          out_sems[b]).wait()


@jax.jit
def _run(dist, grp, tbl):
  mesh = plsc.VectorSubcoreMesh(core_axis_name="c", subcore_axis_name="s")
  return pl.kernel(
      _body,
      out_type=jax.ShapeDtypeStruct((R, C), jnp.float32),
      mesh=mesh,
      compiler_params=pltpu.CompilerParams(needs_layout_passes=False),
      scratch_types=[
          pltpu.VMEM((TBL,), jnp.float32),
          *([pltpu.VMEM((CR, C), jnp.float32)] * NBUF),
          *([pltpu.VMEM((CR, C), jnp.int32)] * NBUF),
          *([pltpu.VMEM((CR, C), jnp.float32)] * NBUF),
          *([pltpu.SemaphoreType.DMA] * (3 * NBUF)),
      ],
  )(dist, grp, tbl)


def kernel(distance_matrix, groups, group_weights):
  tbl = jnp.zeros((TBL,), jnp.float32).at[:group_weights.shape[0]].set(
      group_weights[:, 0])
  return _run(distance_matrix, groups.astype(jnp.int32), tbl)


# speedup vs baseline: 1.1716x; 1.0030x over previous
import jax
import jax.numpy as jnp
from jax import lax
from jax.experimental import pallas as pl
from jax.experimental.pallas import tpu as pltpu
from jax.experimental.pallas import tpu_sc as plsc

R, C = 16384, 1024
NW = 32
ROWS_W = R // NW
CR = 8
NCH = ROWS_W // CR
NBUF = 4
NOUTER = NCH // NBUF
TBL = 128
L = 16
VPC = CR * C // L


def _body(dist_hbm, grp_hbm, tbl_hbm, out_hbm,
          tbl_v, dist_v0, dist_v1, dist_v2, dist_v3,
          grp_v0, grp_v1, grp_v2, grp_v3,
          out_v0, out_v1, out_v2, out_v3,
          sd0, sd1, sd2, sd3, sg0, sg1, sg2, sg3, so0, so1, so2, so3):
  dist_bufs = [dist_v0, dist_v1, dist_v2, dist_v3]
  grp_bufs = [grp_v0, grp_v1, grp_v2, grp_v3]
  out_bufs = [out_v0, out_v1, out_v2, out_v3]
  cid = lax.axis_index("c")
  sid = lax.axis_index("s")
  wid = sid * 2 + cid
  base = wid * ROWS_W

  in_sems_d = [sd0, sd1, sd2, sd3]
  in_sems_g = [sg0, sg1, sg2, sg3]
  out_sems = [so0, so1, so2, so3]

  pltpu.sync_copy(tbl_hbm, tbl_v)

  lanes = lax.iota(jnp.int32, L)

  def in_slices(cg):
    row = base + cg * CR
    return dist_hbm.at[pl.ds(row, CR)], grp_hbm.at[pl.ds(row, CR)]

  def out_slice(cg):
    return out_hbm.at[pl.ds(base + cg * CR, CR)]

  for b in range(NBUF):
    dsl, gsl = in_slices(b)
    pltpu.async_copy(dsl, dist_bufs[b], in_sems_d[b])
    pltpu.async_copy(gsl, grp_bufs[b], in_sems_g[b])

  @pl.loop(0, NOUTER)
  def outer(j):
    for b in range(NBUF):
      cg = j * NBUF + b
      dsl, gsl = in_slices(cg)
      pltpu.make_async_copy(dsl, dist_bufs[b], in_sems_d[b]).wait()
      pltpu.make_async_copy(gsl, grp_bufs[b], in_sems_g[b]).wait()

      @pl.when(j > 0)
      def _():
        pltpu.make_async_copy(out_bufs[b], out_slice((j - 1) * NBUF + b),
                              out_sems[b]).wait()

      db = dist_bufs[b]
      gb = grp_bufs[b]
      ob = out_bufs[b]

      @plsc.parallel_loop(0, VPC, 1, unroll=8)
      def inner(i):
        r = i >> 6
        c = (i & 63) << 4
        g = gb[r, pl.ds(c, L)]
        w = plsc.load_gather(tbl_v, [g])
        d = db[r, pl.ds(c, L)]
        ob[r, pl.ds(c, L)] = jnp.maximum(d * w, 0.0)

      pltpu.async_copy(ob, out_slice(cg), out_sems[b])

      @pl.when(j < NOUTER - 1)
      def _():
        dsl2, gsl2 = in_slices(cg + NBUF)
        pltpu.async_copy(dsl2, dist_bufs[b], in_sems_d[b])
        pltpu.async_copy(gsl2, grp_bufs[b], in_sems_g[b])

  for b in range(NBUF):
    pltpu.make_async_copy(out_bufs[b], out_slice((NOUTER - 1) * NBUF + b),
                          out_sems[b]).wait()


@jax.jit
def _run(dist, grp, tbl):
  mesh = plsc.VectorSubcoreMesh(core_axis_name="c", subcore_axis_name="s")
  return pl.kernel(
      _body,
      out_type=jax.ShapeDtypeStruct((R, C), jnp.float32),
      mesh=mesh,
      compiler_params=pltpu.CompilerParams(needs_layout_passes=False),
      scratch_types=[
          pltpu.VMEM((TBL,), jnp.float32),
          pltpu.VMEM((CR, C), jnp.float32),
          pltpu.VMEM((CR, C), jnp.float32),
          pltpu.VMEM((CR, C), jnp.float32),
          pltpu.VMEM((CR, C), jnp.float32),
          pltpu.VMEM((CR, C), jnp.int32),
          pltpu.VMEM((CR, C), jnp.int32),
          pltpu.VMEM((CR, C), jnp.int32),
          pltpu.VMEM((CR, C), jnp.int32),
          pltpu.VMEM((CR, C), jnp.float32),
          pltpu.VMEM((CR, C), jnp.float32),
          pltpu.VMEM((CR, C), jnp.float32),
          pltpu.VMEM((CR, C), jnp.float32),          pltpu.SemaphoreType.DMA,
          pltpu.SemaphoreType.DMA,
          pltpu.SemaphoreType.DMA,
          pltpu.SemaphoreType.DMA,
          pltpu.SemaphoreType.DMA,
          pltpu.SemaphoreType.DMA,
          pltpu.SemaphoreType.DMA,
          pltpu.SemaphoreType.DMA,
          pltpu.SemaphoreType.DMA,
          pltpu.SemaphoreType.DMA,
          pltpu.SemaphoreType.DMA,
          pltpu.SemaphoreType.DMA,
      ],
  )(dist, grp, tbl)


def kernel(distance_matrix, groups, group_weights):
  tbl = jnp.zeros((TBL,), jnp.float32).at[:group_weights.shape[0]].set(
      group_weights[:, 0])
  return _run(distance_matrix, groups.astype(jnp.int32), tbl)
